# baseline (speedup 1.0000x reference)
import functools

import jax
import jax.numpy as jnp
from jax import lax
from jax.experimental import pallas as pl
from jax.experimental.pallas import tpu as pltpu
from jax.experimental.pallas import tpu_sc as plsc

GAMMA = 0.99

B = 16384
A = 1024
RB = 1024
NBLK = B // RB

NC = 2
NS = 16
NW = NC * NS
PER_W = B // NW
L = 16
CROWS = 32
NCHUNK = PER_W // CROWS


def _argmax_body(av_ref, out_ref):
    av = av_ref[...]
    mx = jnp.max(av, axis=1, keepdims=True)
    cols = lax.broadcasted_iota(jnp.int32, (RB, A), 1)
    big = jnp.int32(2**30)
    cand = jnp.where(av == mx, cols, big)
    out_ref[0, 0, :] = jnp.min(cand, axis=1)


def _argmax_cols(action_values):
    out = pl.pallas_call(
        _argmax_body,
        grid=(NBLK,),
        in_specs=[pl.BlockSpec((RB, A), lambda i: (i, 0))],
        out_specs=pl.BlockSpec((1, 1, RB), lambda i: (i, 0, 0)),
        out_shape=jax.ShapeDtypeStruct((NBLK, 1, RB), jnp.int32),
    )(action_values)
    return out.reshape(B)


def _sc_body(actual_hbm, cidx_hbm, rew_hbm, done_hbm, out_hbm,
             cidx_v, chunk_v, rew_v, done_v, out_v, sem):
    wid = lax.axis_index("s") * NC + lax.axis_index("c")
    base = wid * PER_W
    pltpu.sync_copy(cidx_hbm.at[pl.ds(base, PER_W)], cidx_v)
    pltpu.sync_copy(rew_hbm.at[pl.ds(base, PER_W)], rew_v)
    pltpu.sync_copy(done_hbm.at[pl.ds(base, PER_W)], done_v)
    lanes = lax.iota(jnp.int32, L)

    def chunk_step(k, _):
        pltpu.async_copy(
            actual_hbm.at[pl.ds(base + k * CROWS, CROWS)], chunk_v, sem,
        ).wait()
        for h in range(CROWS // L):
            sl = pl.ds(k * CROWS + h * L, L)
            cvec = cidx_v[sl]
            lr = lanes + h * L
            v = plsc.load_gather(chunk_v, [lr, cvec])
            dn = done_v[sl]
            rw = rew_v[sl]
            w = jnp.where(dn != jnp.float32(0.0), jnp.float32(0.0), v)
            out_v[sl] = w * jnp.float32(GAMMA) + rw
        return _

    lax.fori_loop(0, NCHUNK, chunk_step, 0)
    pltpu.sync_copy(out_v, out_hbm.at[pl.ds(base, PER_W)])


def _sc_select_epilogue(actual, cidx, rew, done_f):
    mesh = plsc.VectorSubcoreMesh(
        core_axis_name="c", subcore_axis_name="s",
        num_cores=NC, num_subcores=NS,
    )
    f = functools.partial(
        pl.kernel,
        mesh=mesh,
        out_type=jax.ShapeDtypeStruct((B,), jnp.float32),
        scratch_types=[
            pltpu.VMEM((PER_W,), jnp.int32),
            pltpu.VMEM((CROWS, A), jnp.float32),
            pltpu.VMEM((PER_W,), jnp.float32),
            pltpu.VMEM((PER_W,), jnp.float32),
            pltpu.VMEM((PER_W,), jnp.float32),
            pltpu.SemaphoreType.DMA,
        ],
        compiler_params=pltpu.CompilerParams(use_tc_tiling_on_sc=True, needs_layout_passes=False),
    )(_sc_body)
    return f(actual, cidx, rew, done_f)


def kernel(next_state_actual_values, next_state_action_values, reward, is_done):
    cidx = _argmax_cols(next_state_action_values)
    rew = reward.reshape(B)
    done_f = is_done.astype(jnp.float32).reshape(B)
    return _sc_select_epilogue(next_state_actual_values, cidx, rew, done_f)


# speedup vs baseline: 1.0283x; 1.0283x over previous
import functools

import jax
import jax.numpy as jnp
from jax import lax
from jax.experimental import pallas as pl
from jax.experimental.pallas import tpu as pltpu
from jax.experimental.pallas import tpu_sc as plsc

GAMMA = 0.99

B = 16384
A = 1024
RB = 1024
G = 4
GROWS = B // G
NBLKG = GROWS // RB

NC = 2
NS = 16
NW = NC * NS
PER_W = GROWS // NW
L = 16
CROWS = 32
NCHUNK = PER_W // CROWS


def _argmax_body(av_ref, out_ref):
    av = av_ref[...]
    mx = jnp.max(av, axis=1, keepdims=True)
    cols = lax.broadcasted_iota(jnp.int32, (RB, A), 1)
    big = jnp.int32(2**30)
    cand = jnp.where(av == mx, cols, big)
    out_ref[0, 0, :] = jnp.min(cand, axis=1)


def _argmax_cols(action_values, g):
    out = pl.pallas_call(
        _argmax_body,
        grid=(NBLKG,),
        in_specs=[pl.BlockSpec((RB, A), lambda i, g=g: (g * NBLKG + i, 0))],
        out_specs=pl.BlockSpec((1, 1, RB), lambda i: (i, 0, 0)),
        out_shape=jax.ShapeDtypeStruct((NBLKG, 1, RB), jnp.int32),
    )(action_values)
    return out.reshape(GROWS)


def _sc_body(g, actual_hbm, cidx_hbm, rew_hbm, done_hbm, out_hbm,
             cidx_v, buf0_v, buf1_v, rew_v, done_v, out_v, sem0, sem1):
    wid = lax.axis_index("s") * NC + lax.axis_index("c")
    lbase = wid * PER_W
    base = g * GROWS + lbase
    pltpu.sync_copy(cidx_hbm.at[pl.ds(lbase, PER_W)], cidx_v)
    pltpu.sync_copy(rew_hbm.at[pl.ds(base, PER_W)], rew_v)
    pltpu.sync_copy(done_hbm.at[pl.ds(base, PER_W)], done_v)
    lanes = lax.iota(jnp.int32, L)

    bufs = [buf0_v, buf1_v]
    sems = [sem0, sem1]
    descs = [None, None]
    descs[0] = pltpu.async_copy(
        actual_hbm.at[pl.ds(base, CROWS)], bufs[0], sems[0])
    for k in range(NCHUNK):
        if k + 1 < NCHUNK:
            descs[(k + 1) % 2] = pltpu.async_copy(
                actual_hbm.at[pl.ds(base + (k + 1) * CROWS, CROWS)],
                bufs[(k + 1) % 2], sems[(k + 1) % 2])
        descs[k % 2].wait()
        buf = bufs[k % 2]
        for h in range(CROWS // L):
            sl = pl.ds(k * CROWS + h * L, L)
            cvec = cidx_v[sl]
            lr = lanes + h * L
            v = plsc.load_gather(buf, [lr, cvec])
            dn = done_v[sl]
            rw = rew_v[sl]
            w = jnp.where(dn != jnp.float32(0.0), jnp.float32(0.0), v)
            out_v[sl] = w * jnp.float32(GAMMA) + rw
    pltpu.sync_copy(out_v, out_hbm.at[pl.ds(lbase, PER_W)])


def _sc_select_epilogue(actual, cidx_g, rew, done_f, g):
    mesh = plsc.VectorSubcoreMesh(
        core_axis_name="c", subcore_axis_name="s",
        num_cores=NC, num_subcores=NS,
    )
    f = functools.partial(
        pl.kernel,
        mesh=mesh,
        out_type=jax.ShapeDtypeStruct((GROWS,), jnp.float32),
        scratch_types=[
            pltpu.VMEM((PER_W,), jnp.int32),
            pltpu.VMEM((CROWS, A), jnp.float32),
            pltpu.VMEM((CROWS, A), jnp.float32),
            pltpu.VMEM((PER_W,), jnp.float32),
            pltpu.VMEM((PER_W,), jnp.float32),
            pltpu.VMEM((PER_W,), jnp.float32),
            pltpu.SemaphoreType.DMA,
            pltpu.SemaphoreType.DMA,
        ],
        compiler_params=pltpu.CompilerParams(
            use_tc_tiling_on_sc=True, needs_layout_passes=False),
    )(functools.partial(_sc_body, g))
    return f(actual, cidx_g, rew, done_f)


def kernel(next_state_actual_values, next_state_action_values, reward, is_done):
    rew = reward.reshape(B)
    done_f = is_done.astype(jnp.float32).reshape(B)
    outs = []
    for g in range(G):
        cidx_g = _argmax_cols(next_state_action_values, g)
        outs.append(_sc_select_epilogue(
            next_state_actual_values, cidx_g, rew, done_f, g))
    return jnp.concatenate(outs)


# speedup vs baseline: 1.0287x; 1.0004x over previous
import functools

import jax
import jax.numpy as jnp
from jax import lax
from jax.experimental import pallas as pl
from jax.experimental.pallas import tpu as pltpu
from jax.experimental.pallas import tpu_sc as plsc

GAMMA = 0.99

B = 16384
A = 1024
RB = 1024
G = 4
GROWS = B // G
NBLKG = GROWS // RB

NC = 2
NS = 16
NW = NC * NS
PER_W = GROWS // NW
L = 16
CROWS = 32
NCHUNK = PER_W // CROWS


def _argmax_body(av_ref, out_ref):
    av = av_ref[...]
    mx = jnp.max(av, axis=1, keepdims=True)
    cols = lax.broadcasted_iota(jnp.int32, (RB, A), 1)
    big = jnp.int32(2**30)
    cand = jnp.where(av == mx, cols, big)
    out_ref[0, 0, :] = jnp.min(cand, axis=1)


def _argmax_cols(action_values, g):
    out = pl.pallas_call(
        _argmax_body,
        grid=(NBLKG,),
        in_specs=[pl.BlockSpec((RB, A), lambda i, g=g: (g * NBLKG + i, 0))],
        out_specs=pl.BlockSpec((1, 1, RB), lambda i: (i, 0, 0)),
        out_shape=jax.ShapeDtypeStruct((NBLKG, 1, RB), jnp.int32),
    )(action_values)
    return out.reshape(GROWS)


def _sc_body(g, actual_hbm, cidx_hbm, rew_hbm, done_hbm, out_hbm,
             cidx_v, buf0_v, buf1_v, rew_v, done_v, out_v, sem0, sem1):
    wid = lax.axis_index("s") * NC + lax.axis_index("c")
    lbase = wid * PER_W
    base = g * GROWS + lbase
    pltpu.sync_copy(cidx_hbm.at[pl.ds(lbase, PER_W)], cidx_v)
    pltpu.sync_copy(rew_hbm.at[pl.ds(base, PER_W)], rew_v)
    pltpu.sync_copy(done_hbm.at[pl.ds(base, PER_W)], done_v)
    lanes = lax.iota(jnp.int32, L)

    bufs = [buf0_v, buf1_v]
    sems = [sem0, sem1]
    descs = [None, None]
    descs[0] = pltpu.async_copy(
        actual_hbm.at[pl.ds(base, CROWS)], bufs[0], sems[0])
    for k in range(NCHUNK):
        if k + 1 < NCHUNK:
            descs[(k + 1) % 2] = pltpu.async_copy(
                actual_hbm.at[pl.ds(base + (k + 1) * CROWS, CROWS)],
                bufs[(k + 1) % 2], sems[(k + 1) % 2])
        descs[k % 2].wait()
        buf = bufs[k % 2]
        for h in range(CROWS // L):
            sl = pl.ds(k * CROWS + h * L, L)
            cvec = cidx_v[sl]
            lr = lanes + h * L
            v = plsc.load_gather(buf, [lr, cvec])
            dn = done_v[sl]
            rw = rew_v[sl]
            w = jnp.where(dn != jnp.float32(0.0), jnp.float32(0.0), v)
            out_v[sl] = w * jnp.float32(GAMMA) + rw
    pltpu.sync_copy(out_v, out_hbm.at[pl.ds(lbase, PER_W)])


def _sc_select_epilogue(actual, cidx_g, rew, done_f, g):
    mesh = plsc.VectorSubcoreMesh(
        core_axis_name="c", subcore_axis_name="s",
        num_cores=NC, num_subcores=NS,
    )
    f = functools.partial(
        pl.kernel,
        mesh=mesh,
        out_type=jax.ShapeDtypeStruct((GROWS,), jnp.float32),
        scratch_types=[
            pltpu.VMEM((PER_W,), jnp.int32),
            pltpu.VMEM((CROWS, A), jnp.float32),
            pltpu.VMEM((CROWS, A), jnp.float32),
            pltpu.VMEM((PER_W,), jnp.float32),
            pltpu.VMEM((PER_W,), jnp.float32),
            pltpu.VMEM((PER_W,), jnp.float32),
            pltpu.SemaphoreType.DMA,
            pltpu.SemaphoreType.DMA,
        ],
        compiler_params=pltpu.CompilerParams(
            use_tc_tiling_on_sc=True, needs_layout_passes=False, skip_device_barrier=True),
    )(functools.partial(_sc_body, g))
    return f(actual, cidx_g, rew, done_f)


def kernel(next_state_actual_values, next_state_action_values, reward, is_done):
    rew = reward.reshape(B)
    done_f = is_done.astype(jnp.float32).reshape(B)
    outs = []
    for g in range(G):
        cidx_g = _argmax_cols(next_state_action_values, g)
        outs.append(_sc_select_epilogue(
            next_state_actual_values, cidx_g, rew, done_f, g))
    return jnp.concatenate(outs)


# speedup vs baseline: 1.0288x; 1.0001x over previous
import functools

import jax
import jax.numpy as jnp
from jax import lax
from jax.experimental import pallas as pl
from jax.experimental.pallas import tpu as pltpu
from jax.experimental.pallas import tpu_sc as plsc

GAMMA = 0.99

B = 16384
A = 1024
RB = 1024
G = 4
GROWS = B // G
NBLKG = GROWS // RB

NC = 2
NS = 16
NW = NC * NS
PER_W = GROWS // NW
L = 16
CROWS = 32
NCHUNK = PER_W // CROWS


def _argmax_body(av_ref, out_ref):
    av = av_ref[...]
    mx = jnp.max(av, axis=1, keepdims=True)
    cols = lax.broadcasted_iota(jnp.int32, (RB, A), 1)
    big = jnp.int32(2**30)
    cand = jnp.where(av == mx, cols, big)
    out_ref[0, 0, :] = jnp.min(cand, axis=1)


def _argmax_cols(action_values, g):
    out = pl.pallas_call(
        _argmax_body,
        grid=(NBLKG,),
        in_specs=[pl.BlockSpec((RB, A), lambda i, g=g: (g * NBLKG + i, 0))],
        out_specs=pl.BlockSpec((1, 1, RB), lambda i: (i, 0, 0)),
        out_shape=jax.ShapeDtypeStruct((NBLKG, 1, RB), jnp.int32),
    )(action_values)
    return out.reshape(GROWS)


def _sc_body(g, actual_hbm, cidx_hbm, rew_hbm, done_hbm, out_hbm,
             cidx_v, buf0_v, buf1_v, rew_v, done_v, out_v, sem0, sem1):
    wid = lax.axis_index("s") * NC + lax.axis_index("c")
    lbase = wid * PER_W
    base = g * GROWS + lbase
    pltpu.sync_copy(cidx_hbm.at[pl.ds(lbase, PER_W)], cidx_v)
    pltpu.sync_copy(rew_hbm.at[pl.ds(base, PER_W)], rew_v)
    pltpu.sync_copy(done_hbm.at[pl.ds(base, PER_W)], done_v)
    lanes = lax.iota(jnp.int32, L)

    bufs = [buf0_v, buf1_v]
    sems = [sem0, sem1]
    descs = [None, None]
    descs[0] = pltpu.async_copy(
        actual_hbm.at[pl.ds(base, CROWS)], bufs[0], sems[0])
    for k in range(NCHUNK):
        if k + 1 < NCHUNK:
            descs[(k + 1) % 2] = pltpu.async_copy(
                actual_hbm.at[pl.ds(base + (k + 1) * CROWS, CROWS)],
                bufs[(k + 1) % 2], sems[(k + 1) % 2])
        descs[k % 2].wait()
        buf = bufs[k % 2]
        for h in range(CROWS // L):
            sl = pl.ds(k * CROWS + h * L, L)
            cvec = cidx_v[sl]
            lr = lanes + h * L
            v = plsc.load_gather(buf, [lr, cvec])
            dn = done_v[sl]
            rw = rew_v[sl]
            w = jnp.where(dn != jnp.float32(0.0), jnp.float32(0.0), v)
            out_v[sl] = w * jnp.float32(GAMMA) + rw
    pltpu.sync_copy(out_v, out_hbm.at[pl.ds(lbase, PER_W)])


def _sc_select_epilogue(actual, cidx_g, rew, done_f, g):
    mesh = plsc.VectorSubcoreMesh(
        core_axis_name="c", subcore_axis_name="s",
        num_cores=NC, num_subcores=NS,
    )
    f = functools.partial(
        pl.kernel,
        mesh=mesh,
        out_type=jax.ShapeDtypeStruct((GROWS,), jnp.float32),
        scratch_types=[
            pltpu.VMEM((PER_W,), jnp.int32),
            pltpu.VMEM((CROWS, A), jnp.float32),
            pltpu.VMEM((CROWS, A), jnp.float32),
            pltpu.VMEM((PER_W,), jnp.float32),
            pltpu.VMEM((PER_W,), jnp.float32),
            pltpu.VMEM((PER_W,), jnp.float32),
            pltpu.SemaphoreType.DMA,
            pltpu.SemaphoreType.DMA,
        ],
        compiler_params=pltpu.CompilerParams(
            use_tc_tiling_on_sc=True, needs_layout_passes=False, skip_device_barrier=True, has_side_effects=False),
    )(functools.partial(_sc_body, g))
    return f(actual, cidx_g, rew, done_f)


def kernel(next_state_actual_values, next_state_action_values, reward, is_done):
    rew = reward.reshape(B)
    done_f = is_done.astype(jnp.float32).reshape(B)
    outs = []
    for g in range(G):
        cidx_g = _argmax_cols(next_state_action_values, g)
        outs.append(_sc_select_epilogue(
            next_state_actual_values, cidx_g, rew, done_f, g))
    return jnp.concatenate(outs)
